# baseline (speedup 1.0000x reference)
import jax
import jax.numpy as jnp
from jax import lax
from jax.experimental import pallas as pl
from jax.experimental.pallas import tpu as pltpu
from jax.experimental.pallas import tpu_sc as plsc

N = 10000
E = 320000
D = 128

NC = 2
NS = 16
NW = NC * NS
EPT = E // NW
CHUNK = 128
NCHUNK = 78
TAIL = EPT - NCHUNK * CHUNK
RCHUNK = 80
NRCHUNK = N // RCHUNK
SHIFT = 14
MASK = (1 << SHIFT) - 1


def _mp_kernel(h_hbm, idx_hbm, out_hbm,
               ibuf_a, ibuf_b, sidx_a, didx_a, sidx_b, didx_b,
               bf_a, bf_b, f_a, f_b, acc,
               semi_a, semi_b, semg_a, semg_b):
    c = lax.axis_index("c")
    s = lax.axis_index("s")
    wid = c * NS + s
    e0 = pl.multiple_of(wid * EPT, 8)

    def zero_full(i, carry):
        for j in range(D // 16):
            f_a[i, pl.ds(j * 16, 16)] = jnp.zeros((16,), jnp.float32)
        return carry
    lax.fori_loop(0, CHUNK, zero_full, 0, unroll=4)

    nmine = (NRCHUNK - s + NS - 1) // NS

    def zero_acc(i, carry):
        r0 = pl.multiple_of((s + i * NS) * RCHUNK, 8)
        pltpu.sync_copy(f_a.at[pl.ds(0, RCHUNK), :],
                        acc.at[pl.ds(r0, RCHUNK), :])
        return carry

    lax.fori_loop(0, nmine, zero_acc, 0)
    plsc.subcore_barrier()

    def unpidx(ibuf, sdst, ddst):
        for v in range(CHUNK // 16):
            p = ibuf[pl.ds(v * 16, 16)]
            sdst[pl.ds(v * 16, 16)] = p & MASK
            ddst[pl.ds(v * 16, 16)] = lax.shift_right_logical(p, SHIFT)

    def conv(bf, f):
        def body(r, carry):
            for q in range(4):
                x = bf[r, pl.ds(16 * q, 16)]
                xb = plsc.bitcast(x, jnp.bfloat16)
                a, b = plsc.unpack(xb, format=plsc.PackFormat.INTERLEAVED,
                                   preferred_element_type=jnp.float32)
                f[r, pl.ds(32 * q, 16)] = a
                f[r, pl.ds(32 * q + 16, 16)] = b
            return carry
        lax.fori_loop(0, CHUNK, body, 0, unroll=4)

    pltpu.async_copy(idx_hbm.at[pl.ds(e0, CHUNK)], ibuf_a, semi_a)
    pltpu.async_copy(idx_hbm.at[pl.ds(e0 + CHUNK, CHUNK)], ibuf_b, semi_b)
    pltpu.make_async_copy(idx_hbm.at[pl.ds(e0, CHUNK)], ibuf_a, semi_a).wait()
    unpidx(ibuf_a, sidx_a, didx_a)
    pltpu.async_copy(h_hbm.at[sidx_a], bf_a, semg_a)

    def pipe(j, carry):
        c0 = 2 * j
        pltpu.make_async_copy(idx_hbm.at[pl.ds(e0, CHUNK)], ibuf_b,
                              semi_b).wait()
        unpidx(ibuf_b, sidx_b, didx_b)
        pltpu.async_copy(h_hbm.at[sidx_b], bf_b, semg_b)
        pltpu.async_copy(idx_hbm.at[pl.ds(e0 + (c0 + 2) * CHUNK, CHUNK)],
                         ibuf_a, semi_a)
        pltpu.make_async_copy(h_hbm.at[sidx_a], bf_a, semg_a).wait()
        conv(bf_a, f_a)
        pltpu.sync_copy(f_a, acc.at[didx_a], add=True)
        pltpu.make_async_copy(idx_hbm.at[pl.ds(e0, CHUNK)], ibuf_a,
                              semi_a).wait()
        unpidx(ibuf_a, sidx_a, didx_a)
        pltpu.async_copy(h_hbm.at[sidx_a], bf_a, semg_a)
        pltpu.async_copy(idx_hbm.at[pl.ds(e0 + (c0 + 3) * CHUNK, CHUNK)],
                         ibuf_b, semi_b)
        pltpu.make_async_copy(h_hbm.at[sidx_b], bf_b, semg_b).wait()
        conv(bf_b, f_b)
        pltpu.sync_copy(f_b, acc.at[didx_b], add=True)
        return carry

    lax.fori_loop(0, NCHUNK // 2 - 1, pipe, 0)
    pltpu.make_async_copy(idx_hbm.at[pl.ds(e0, CHUNK)], ibuf_b, semi_b).wait()
    unpidx(ibuf_b, sidx_b, didx_b)
    pltpu.async_copy(h_hbm.at[sidx_b], bf_b, semg_b)
    pltpu.make_async_copy(h_hbm.at[sidx_a], bf_a, semg_a).wait()
    conv(bf_a, f_a)
    pltpu.sync_copy(f_a, acc.at[didx_a], add=True)
    pltpu.async_copy(idx_hbm.at[pl.ds(e0 + NCHUNK * CHUNK, TAIL)],
                     ibuf_a.at[pl.ds(0, TAIL)], semi_a)
    pltpu.make_async_copy(h_hbm.at[sidx_b], bf_b, semg_b).wait()
    conv(bf_b, f_b)
    pltpu.sync_copy(f_b, acc.at[didx_b], add=True)
    pltpu.make_async_copy(idx_hbm.at[pl.ds(e0, TAIL)],
                          ibuf_a.at[pl.ds(0, TAIL)], semi_a).wait()
    p = ibuf_a[pl.ds(0, TAIL)]
    sidx_a[pl.ds(0, TAIL)] = p & MASK
    didx_a[pl.ds(0, TAIL)] = lax.shift_right_logical(p, SHIFT)
    pltpu.async_copy(h_hbm.at[sidx_a.at[pl.ds(0, TAIL)]],
                     bf_a.at[pl.ds(0, TAIL), :], semg_a).wait()

    def tconv(r, carry):
        for q in range(4):
            x = bf_a[r, pl.ds(16 * q, 16)]
            xb = plsc.bitcast(x, jnp.bfloat16)
            a, b = plsc.unpack(xb, format=plsc.PackFormat.INTERLEAVED,
                               preferred_element_type=jnp.float32)
            f_a[r, pl.ds(32 * q, 16)] = a
            f_a[r, pl.ds(32 * q + 16, 16)] = b
        return carry
    lax.fori_loop(0, TAIL, tconv, 0)
    pltpu.sync_copy(f_a.at[pl.ds(0, TAIL), :],
                    acc.at[didx_a.at[pl.ds(0, TAIL)]], add=True)

    plsc.subcore_barrier()

    def wout(i, carry):
        r0 = pl.multiple_of((s + i * NS) * RCHUNK, 8)
        pltpu.sync_copy(acc.at[pl.ds(r0, RCHUNK), :],
                        out_hbm.at[c, pl.ds(r0, RCHUNK), :])
        return carry

    lax.fori_loop(0, nmine, wout, 0)


def _message_pass(h, idx):
    mesh = plsc.VectorSubcoreMesh(core_axis_name="c", subcore_axis_name="s",
                                  num_cores=NC, num_subcores=NS)
    return pl.kernel(
        _mp_kernel,
        out_type=jax.ShapeDtypeStruct((NC, N, D), jnp.float32),
        mesh=mesh,
        compiler_params=pltpu.CompilerParams(use_tc_tiling_on_sc=False,
                                            needs_layout_passes=False),
        scratch_types=[
            pltpu.VMEM((CHUNK,), jnp.int32),
            pltpu.VMEM((CHUNK,), jnp.int32),
            pltpu.VMEM((CHUNK,), jnp.int32),
            pltpu.VMEM((CHUNK,), jnp.int32),
            pltpu.VMEM((CHUNK,), jnp.int32),
            pltpu.VMEM((CHUNK,), jnp.int32),
            pltpu.VMEM((CHUNK, D // 2), jnp.uint32),
            pltpu.VMEM((CHUNK, D // 2), jnp.uint32),
            pltpu.VMEM((CHUNK, D), jnp.float32),
            pltpu.VMEM((CHUNK, D), jnp.float32),
            pltpu.VMEM_SHARED((N, D), jnp.float32),
            pltpu.SemaphoreType.DMA,
            pltpu.SemaphoreType.DMA,
            pltpu.SemaphoreType.DMA,
            pltpu.SemaphoreType.DMA,
        ],
    )(h, idx)


ROWB = 2000


def _pack_bf16_pairs(r):
    def bfbits(x):
        u = lax.bitcast_convert_type(x, jnp.uint32)
        return (u + 0x7FFF + ((u >> 16) & 1)) >> 16
    a = bfbits(r[:, :D // 2])
    b = bfbits(r[:, D // 2:])
    return a | (b << 16)


def _mm_kernel(x_ref, w_ref, o_ref):
    r = jnp.dot(x_ref[...], w_ref[...], preferred_element_type=jnp.float32)
    o_ref[...] = _pack_bf16_pairs(r)


def _matmul(x, w):
    return pl.pallas_call(
        _mm_kernel,
        grid=(N // ROWB,),
        in_specs=[
            pl.BlockSpec((ROWB, D), lambda i: (i, 0)),
            pl.BlockSpec((D, D), lambda i: (0, 0)),
        ],
        out_specs=pl.BlockSpec((ROWB, D // 2), lambda i: (i, 0)),
        out_shape=jax.ShapeDtypeStruct((N, D // 2), jnp.uint32),
    )(x, w)


def _fuse_kernel(a_ref, b_ref, w_ref, o_ref):
    z = a_ref[0] + a_ref[1] + b_ref[...]
    z = jnp.maximum(z, 0.0)
    r = jnp.dot(z, w_ref[...], preferred_element_type=jnp.float32)
    o_ref[...] = _pack_bf16_pairs(r)


def _relu_matmul(parts, b, w):
    return pl.pallas_call(
        _fuse_kernel,
        grid=(N // ROWB,),
        in_specs=[
            pl.BlockSpec((NC, ROWB, D), lambda i: (0, i, 0)),
            pl.BlockSpec((1, D), lambda i: (0, 0)),
            pl.BlockSpec((D, D), lambda i: (0, 0)),
        ],
        out_specs=pl.BlockSpec((ROWB, D // 2), lambda i: (i, 0)),
        out_shape=jax.ShapeDtypeStruct((N, D // 2), jnp.uint32),
    )(parts, b.reshape(1, D), w)


def _lsm_kernel(a_ref, b_ref, o_ref):
    t = a_ref[0] + a_ref[1] + b_ref[...]
    m = jnp.max(t, axis=-1, keepdims=True)
    e = jnp.exp(t - m)
    lse = jnp.log(jnp.sum(e, axis=-1, keepdims=True)) + m
    o_ref[...] = t - lse


def _log_softmax(parts, b):
    return pl.pallas_call(
        _lsm_kernel,
        grid=(N // ROWB,),
        in_specs=[
            pl.BlockSpec((NC, ROWB, D), lambda i: (0, i, 0)),
            pl.BlockSpec((1, D), lambda i: (0, 0)),
        ],
        out_specs=pl.BlockSpec((ROWB, D), lambda i: (i, 0)),
        out_shape=jax.ShapeDtypeStruct((N, D), jnp.float32),
    )(parts, b.reshape(1, D))


PERM2 = tuple(32 * (w // 16) + (w % 16) for w in range(64)) + tuple(
    32 * (w // 16) + 16 + (w % 16) for w in range(64))


def kernel(x, edge_index, W1, b1, W2, b2, W3, b3):
    ei = edge_index.astype(jnp.int32)
    idx = ei[0] | (ei[1] << SHIFT)

    p = jnp.array(PERM2, dtype=jnp.int32)

    h = _matmul(x, W1[:, p])
    parts = _message_pass(h, idx)
    h = _relu_matmul(parts, b1, W2[:, p])
    parts = _message_pass(h, idx)
    h = _relu_matmul(parts, b2, W3[:, p])
    parts = _message_pass(h, idx)
    return _log_softmax(parts, b3)


# speedup vs baseline: 2.2525x; 2.2525x over previous
import jax
import jax.numpy as jnp
from jax import lax
from jax.experimental import pallas as pl
from jax.experimental.pallas import tpu as pltpu
from jax.experimental.pallas import tpu_sc as plsc

N = 10000
E = 320000
D = 128

NC = 2
NS = 16
NW = NC * NS
EPT = E // NW
CHUNK = 128
NCHUNK = 78
TAIL = EPT - NCHUNK * CHUNK
RCHUNK = 80
NRCHUNK = N // RCHUNK
SHIFT = 14
MASK = (1 << SHIFT) - 1


def _mp_kernel(h_hbm, idx_hbm, out_hbm,
               pidx, sidx_a, didx_a, sidx_b, didx_b,
               rows_a, rows_b, acc, sem_a, sem_b):
    c = lax.axis_index("c")
    s = lax.axis_index("s")
    wid = c * NS + s

    e0 = pl.multiple_of(wid * EPT, 8)
    ld_p = pltpu.async_copy(idx_hbm.at[pl.ds(e0, EPT)], pidx, sem_a)

    def zero_full(i, carry):
        for j in range(D // 16):
            rows_a[i, pl.ds(j * 16, 16)] = jnp.zeros((16,), jnp.float32)
        return carry
    lax.fori_loop(0, CHUNK, zero_full, 0, unroll=4)

    nmine = (NRCHUNK - s + NS - 1) // NS

    def zero_acc(i, carry):
        r0 = pl.multiple_of((s + i * NS) * RCHUNK, 8)
        pltpu.sync_copy(rows_a.at[pl.ds(0, RCHUNK), :],
                        acc.at[pl.ds(r0, RCHUNK), :])
        return carry

    lax.fori_loop(0, nmine, zero_acc, 0)

    ld_p.wait()
    plsc.subcore_barrier()

    def unpack(k, sdst, ddst):
        for v in range(CHUNK // 16):
            p = pidx[pl.ds(k * CHUNK + v * 16, 16)]
            sdst[pl.ds(v * 16, 16)] = p & MASK
            ddst[pl.ds(v * 16, 16)] = lax.shift_right_logical(p, SHIFT)

    unpack(0, sidx_a, didx_a)
    pltpu.async_copy(h_hbm.at[sidx_a], rows_a, sem_a)

    def pipe(j, carry):
        c0 = 2 * j
        unpack(c0 + 1, sidx_b, didx_b)
        gb = pltpu.async_copy(h_hbm.at[sidx_b], rows_b, sem_b)
        pltpu.make_async_copy(h_hbm.at[sidx_a], rows_a, sem_a).wait()
        pltpu.sync_copy(rows_a, acc.at[didx_a], add=True)
        unpack(c0 + 2, sidx_a, didx_a)
        pltpu.async_copy(h_hbm.at[sidx_a], rows_a, sem_a)
        gb.wait()
        pltpu.sync_copy(rows_b, acc.at[didx_b], add=True)
        return carry

    lax.fori_loop(0, NCHUNK // 2 - 1, pipe, 0)
    unpack(NCHUNK - 1, sidx_b, didx_b)
    gb = pltpu.async_copy(h_hbm.at[sidx_b], rows_b, sem_b)
    pltpu.make_async_copy(h_hbm.at[sidx_a], rows_a, sem_a).wait()
    pltpu.sync_copy(rows_a, acc.at[didx_a], add=True)
    p = pidx[pl.ds(NCHUNK * CHUNK, TAIL)]
    sidx_a[pl.ds(0, TAIL)] = p & MASK
    didx_a[pl.ds(0, TAIL)] = lax.shift_right_logical(p, SHIFT)
    gt = pltpu.async_copy(h_hbm.at[sidx_a.at[pl.ds(0, TAIL)]],
                          rows_a.at[pl.ds(0, TAIL), :], sem_a)
    gb.wait()
    pltpu.sync_copy(rows_b, acc.at[didx_b], add=True)
    gt.wait()
    pltpu.sync_copy(rows_a.at[pl.ds(0, TAIL), :],
                    acc.at[didx_a.at[pl.ds(0, TAIL)]], add=True)

    plsc.subcore_barrier()

    def wout(i, carry):
        r0 = pl.multiple_of((s + i * NS) * RCHUNK, 8)
        pltpu.sync_copy(acc.at[pl.ds(r0, RCHUNK), :],
                        out_hbm.at[c, pl.ds(r0, RCHUNK), :])
        return carry

    lax.fori_loop(0, nmine, wout, 0)


def _message_pass(h, idx):
    mesh = plsc.VectorSubcoreMesh(core_axis_name="c", subcore_axis_name="s",
                                  num_cores=NC, num_subcores=NS)
    return pl.kernel(
        _mp_kernel,
        out_type=jax.ShapeDtypeStruct((NC, N, D), jnp.float32),
        mesh=mesh,
        compiler_params=pltpu.CompilerParams(use_tc_tiling_on_sc=False),
        scratch_types=[
            pltpu.VMEM((EPT,), jnp.int32),
            pltpu.VMEM((CHUNK,), jnp.int32),
            pltpu.VMEM((CHUNK,), jnp.int32),
            pltpu.VMEM((CHUNK,), jnp.int32),
            pltpu.VMEM((CHUNK,), jnp.int32),
            pltpu.VMEM((CHUNK, D), jnp.float32),
            pltpu.VMEM((CHUNK, D), jnp.float32),
            pltpu.VMEM_SHARED((N, D), jnp.float32),
            pltpu.SemaphoreType.DMA,
            pltpu.SemaphoreType.DMA,
        ],
    )(h, idx)


ROWB = 2000


def _mm_kernel(x_ref, w_ref, o_ref):
    o_ref[...] = jnp.dot(x_ref[...], w_ref[...],
                         preferred_element_type=jnp.float32)


def _matmul(x, w):
    return pl.pallas_call(
        _mm_kernel,
        grid=(N // ROWB,),
        in_specs=[
            pl.BlockSpec((ROWB, D), lambda i: (i, 0)),
            pl.BlockSpec((D, D), lambda i: (0, 0)),
        ],
        out_specs=pl.BlockSpec((ROWB, D), lambda i: (i, 0)),
        out_shape=jax.ShapeDtypeStruct((N, D), jnp.float32),
    )(x, w)


def _fuse_kernel(a_ref, b_ref, w_ref, o_ref):
    z = a_ref[0] + a_ref[1] + b_ref[...]
    z = jnp.maximum(z, 0.0)
    o_ref[...] = jnp.dot(z, w_ref[...], preferred_element_type=jnp.float32)


def _relu_matmul(parts, b, w):
    return pl.pallas_call(
        _fuse_kernel,
        grid=(N // ROWB,),
        in_specs=[
            pl.BlockSpec((NC, ROWB, D), lambda i: (0, i, 0)),
            pl.BlockSpec((1, D), lambda i: (0, 0)),
            pl.BlockSpec((D, D), lambda i: (0, 0)),
        ],
        out_specs=pl.BlockSpec((ROWB, D), lambda i: (i, 0)),
        out_shape=jax.ShapeDtypeStruct((N, D), jnp.float32),
    )(parts, b.reshape(1, D), w)


def _lsm_kernel(a_ref, b_ref, o_ref):
    t = a_ref[0] + a_ref[1] + b_ref[...]
    m = jnp.max(t, axis=-1, keepdims=True)
    e = jnp.exp(t - m)
    lse = jnp.log(jnp.sum(e, axis=-1, keepdims=True)) + m
    o_ref[...] = t - lse


def _log_softmax(parts, b):
    return pl.pallas_call(
        _lsm_kernel,
        grid=(N // ROWB,),
        in_specs=[
            pl.BlockSpec((NC, ROWB, D), lambda i: (0, i, 0)),
            pl.BlockSpec((1, D), lambda i: (0, 0)),
        ],
        out_specs=pl.BlockSpec((ROWB, D), lambda i: (i, 0)),
        out_shape=jax.ShapeDtypeStruct((N, D), jnp.float32),
    )(parts, b.reshape(1, D))


def kernel(x, edge_index, W1, b1, W2, b2, W3, b3):
    ei = edge_index.astype(jnp.int32)
    idx = ei[0] | (ei[1] << SHIFT)

    h = _matmul(x, W1)
    parts = _message_pass(h, idx)
    h = _relu_matmul(parts, b1, W2)
    parts = _message_pass(h, idx)
    h = _relu_matmul(parts, b2, W3)
    parts = _message_pass(h, idx)
    return _log_softmax(parts, b3)
